# baseline (speedup 1.0000x reference)
import functools

import jax
import jax.numpy as jnp
from jax import lax
from jax.experimental import pallas as pl
from jax.experimental.pallas import tpu as pltpu
from jax.experimental.pallas import tpu_sc as plsc

NC = 2
NS = 16
NW = NC * NS
CHUNK = 128
NBUF = 2
PW = 256


def _transpose_panel(pbuf, cbuf, b, width, D, rowvecs):
    @pl.loop(0, width)
    def col(i):
        cvec = lax.broadcast(i, (16,))
        half = (i & 1) * D
        row = lax.shift_right_logical(i, 1)
        for p in range(D // 16):
            v = plsc.load_gather(pbuf.at[b], [rowvecs[p], cvec])
            cbuf[b, row, pl.ds(half + p * 16, 16)] = v


def kernel(input_, weight):
    B, S = input_.shape
    V, D = weight.shape
    total = B * S
    assert total % (NW * CHUNK) == 0
    n_chunks = total // (NW * CHUNK)
    n_rounds = n_chunks // NBUF

    idx = input_.reshape(NW, n_chunks, CHUNK).astype(jnp.int32)
    wT = weight.T

    n_full = (V // PW) // NW * NW
    k_per_tile = n_full // NW
    tail0 = n_full * PW

    mesh = plsc.VectorSubcoreMesh(
        core_axis_name="c", subcore_axis_name="s", num_cores=NC, num_subcores=NS
    )

    @functools.partial(
        pl.kernel,
        out_type=jax.ShapeDtypeStruct((V // 2, 2 * D), jnp.float32),
        mesh=mesh,
        scratch_types=[
            pltpu.VMEM((2, D, PW), jnp.float32),
            pltpu.VMEM((2, PW // 2, 2 * D), jnp.float32),
            pltpu.SemaphoreType.DMA((2,)),
            pltpu.SemaphoreType.DMA((2,)),
        ],
        compiler_params=pltpu.CompilerParams(
            use_tc_tiling_on_sc=True, needs_layout_passes=False
        ),
    )
    def compact(wT_hbm, wc_hbm, pbuf, cbuf, psem, wsem):
        wid = lax.axis_index("s") * NC + lax.axis_index("c")
        rowvecs = [lax.iota(jnp.int32, 16) + 16 * p for p in range(D // 16)]

        def col0(k):
            return pl.multiple_of((wid + 32 * k) * PW, PW)

        def sup0(k):
            return pl.multiple_of((wid + 32 * k) * (PW // 2), PW // 2)

        def load(k, b):
            pltpu.async_copy(
                wT_hbm.at[:, pl.ds(col0(k), PW)], pbuf.at[b], psem.at[b]
            )

        def wait_load(k, b):
            pltpu.make_async_copy(
                wT_hbm.at[:, pl.ds(col0(k), PW)], pbuf.at[b], psem.at[b]
            ).wait()

        def store(k, b):
            pltpu.async_copy(
                cbuf.at[b],
                wc_hbm.at[pl.ds(sup0(k), PW // 2)],
                wsem.at[b],
            )

        def wait_store(k, b):
            pltpu.make_async_copy(
                cbuf.at[b],
                wc_hbm.at[pl.ds(sup0(k), PW // 2)],
                wsem.at[b],
            ).wait()

        load(0, 0)

        @pl.loop(0, k_per_tile // 2)
        def body(jj):
            for koff in range(2):
                b = koff
                k = 2 * jj + koff
                wait_load(k, b)

                @pl.when(k < k_per_tile - 1)
                def _():
                    load(k + 1, 1 - b)

                @pl.when(jj > 0)
                def _():
                    wait_store(k - 2, b)

                _transpose_panel(pbuf, cbuf, b, PW, D, rowvecs)
                store(k, b)

        wait_store(k_per_tile - 2, 0)
        wait_store(k_per_tile - 1, 1)

        @pl.when(wid < 5)
        def _():
            c0 = pl.multiple_of(tail0 + 128 * wid, 128)
            s0 = pl.multiple_of(tail0 // 2 + 64 * wid, 64)
            pltpu.async_copy(
                wT_hbm.at[:, pl.ds(c0, 128)], pbuf.at[0, :, pl.ds(0, 128)],
                psem.at[0],
            )
            pltpu.make_async_copy(
                wT_hbm.at[:, pl.ds(c0, 128)], pbuf.at[0, :, pl.ds(0, 128)],
                psem.at[0],
            ).wait()

            def tail_piece(width):
                @pl.loop(0, width)
                def col(i):
                    cvec = lax.broadcast(i, (16,))
                    half = (i & 1) * D
                    row = lax.shift_right_logical(i, 1)
                    for p in range(D // 16):
                        v = plsc.load_gather(pbuf.at[0], [rowvecs[p], cvec])
                        cbuf[0, row, pl.ds(half + p * 16, 16)] = v

                pltpu.async_copy(
                    cbuf.at[0, pl.ds(0, width // 2)],
                    wc_hbm.at[pl.ds(s0, width // 2)],
                    wsem.at[0],
                )
                pltpu.make_async_copy(
                    cbuf.at[0, pl.ds(0, width // 2)],
                    wc_hbm.at[pl.ds(s0, width // 2)],
                    wsem.at[0],
                ).wait()

            @pl.when(wid < 4)
            def _():
                tail_piece(128)

            @pl.when(wid == 4)
            def _():
                tail_piece(64)

    @functools.partial(
        pl.kernel,
        out_type=jax.ShapeDtypeStruct(
            (NW * n_chunks, CHUNK // 2, 2 * D), jnp.float32
        ),
        mesh=mesh,
        scratch_types=[
            pltpu.VMEM((n_chunks, CHUNK), jnp.int32),
            pltpu.VMEM((NBUF, CHUNK), jnp.int32),
            pltpu.VMEM((NBUF, CHUNK, 2 * D), jnp.float32),
            pltpu.VMEM((NBUF, CHUNK // 2, 2 * D), jnp.float32),
            pltpu.SemaphoreType.DMA((NBUF,)),
            pltpu.SemaphoreType.DMA((NBUF,)),
        ],
        compiler_params=pltpu.CompilerParams(use_tc_tiling_on_sc=True),
    )
    def emb(idx_hbm, w_hbm, out_hbm, idx_v, sup_v, super_v, stage_v, gsem, wsem):
        wid = lax.axis_index("s") * NC + lax.axis_index("c")
        rbase = wid * n_chunks
        pltpu.sync_copy(idx_hbm.at[wid], idx_v)

        def compute_sup(c, b):
            for q in range(CHUNK // 16):
                sup_v[b, pl.ds(q * 16, 16)] = lax.shift_right_logical(
                    idx_v[c, pl.ds(q * 16, 16)], 1
                )

        for b in range(NBUF):
            compute_sup(b, b)
            pltpu.async_copy(w_hbm.at[sup_v.at[b]], super_v.at[b], gsem.at[b])

        @pl.loop(0, n_rounds)
        def body(g):
            for b in range(NBUF):
                cur = g * NBUF + b
                pltpu.make_async_copy(
                    w_hbm.at[sup_v.at[b]], super_v.at[b], gsem.at[b]
                ).wait()

                @pl.when(g > 0)
                def _():
                    pltpu.make_async_copy(
                        stage_v.at[b], out_hbm.at[rbase + cur - NBUF], wsem.at[b]
                    ).wait()

                @pl.loop(0, CHUNK // 16)
                def ext(k16):
                    hv = idx_v[cur, pl.ds(k16 * 16, 16)] & 1
                    for l in range(16):
                        hbase = hv[l] * D
                        k = k16 * 16 + l
                        k2, e = k16 * 8 + l // 2, l % 2
                        for p in range(D // 16):
                            stage_v[b, k2, pl.ds(e * D + p * 16, 16)] = (
                                super_v[b, k, pl.ds(hbase + p * 16, 16)]
                            )

                pltpu.async_copy(
                    stage_v.at[b], out_hbm.at[rbase + cur], wsem.at[b]
                )

                @pl.when(cur + NBUF < n_chunks)
                def _():
                    compute_sup(cur + NBUF, b)
                    pltpu.async_copy(
                        w_hbm.at[sup_v.at[b]], super_v.at[b], gsem.at[b]
                    )

        for b in range(NBUF):
            cur = n_chunks - NBUF + b
            pltpu.make_async_copy(
                stage_v.at[b], out_hbm.at[rbase + cur], wsem.at[b]
            ).wait()

    wc = compact(wT)
    out = emb(idx, wc)
    return out.reshape(B, S, D)


# speedup vs baseline: 1.1906x; 1.1906x over previous
import functools

import jax
import jax.numpy as jnp
from jax import lax
from jax.experimental import pallas as pl
from jax.experimental.pallas import tpu as pltpu
from jax.experimental.pallas import tpu_sc as plsc

NC = 2
NS = 16
NW = NC * NS
CHUNK = 128
NBUF = 2
PW = 256


def _transpose_panel(pbuf, cbuf, b, width, D, iot):
    @pl.loop(0, width // 16)
    def grp(g):
        lo = g * 16
        colsv = lax.broadcast(lo, (16,)) + iot
        rows16 = lax.shift_right_logical(colsv, 1)
        colbase = (colsv & 1) * D
        for d in range(D):
            v = pbuf[b, d, pl.ds(lo, 16)]
            plsc.store_scatter(cbuf.at[b], [rows16, colbase + d], v)


def kernel(input_, weight):
    B, S = input_.shape
    V, D = weight.shape
    total = B * S
    assert total % (NW * CHUNK) == 0
    n_chunks = total // (NW * CHUNK)
    n_rounds = n_chunks // NBUF

    idx = input_.reshape(NW, n_chunks, CHUNK).astype(jnp.int32)
    wT = weight.T

    n_full = (V // PW) // NW * NW
    k_per_tile = n_full // NW
    tail0 = n_full * PW

    mesh = plsc.VectorSubcoreMesh(
        core_axis_name="c", subcore_axis_name="s", num_cores=NC, num_subcores=NS
    )

    @functools.partial(
        pl.kernel,
        out_type=jax.ShapeDtypeStruct((V // 2, 2 * D), jnp.float32),
        mesh=mesh,
        scratch_types=[
            pltpu.VMEM((2, D, PW), jnp.float32),
            pltpu.VMEM((2, PW // 2, 2 * D), jnp.float32),
            pltpu.SemaphoreType.DMA((2,)),
            pltpu.SemaphoreType.DMA((2,)),
        ],
        compiler_params=pltpu.CompilerParams(
            use_tc_tiling_on_sc=True, needs_layout_passes=False
        ),
    )
    def compact(wT_hbm, wc_hbm, pbuf, cbuf, psem, wsem):
        wid = lax.axis_index("s") * NC + lax.axis_index("c")
        iot = lax.iota(jnp.int32, 16)

        def col0(k):
            return pl.multiple_of((wid + 32 * k) * PW, PW)

        def sup0(k):
            return pl.multiple_of((wid + 32 * k) * (PW // 2), PW // 2)

        def load(k, b):
            pltpu.async_copy(
                wT_hbm.at[:, pl.ds(col0(k), PW)], pbuf.at[b], psem.at[b]
            )

        def wait_load(k, b):
            pltpu.make_async_copy(
                wT_hbm.at[:, pl.ds(col0(k), PW)], pbuf.at[b], psem.at[b]
            ).wait()

        def store(k, b):
            pltpu.async_copy(
                cbuf.at[b],
                wc_hbm.at[pl.ds(sup0(k), PW // 2)],
                wsem.at[b],
            )

        def wait_store(k, b):
            pltpu.make_async_copy(
                cbuf.at[b],
                wc_hbm.at[pl.ds(sup0(k), PW // 2)],
                wsem.at[b],
            ).wait()

        load(0, 0)

        @pl.loop(0, k_per_tile // 2)
        def body(jj):
            for koff in range(2):
                b = koff
                k = 2 * jj + koff
                wait_load(k, b)

                @pl.when(k < k_per_tile - 1)
                def _():
                    load(k + 1, 1 - b)

                @pl.when(jj > 0)
                def _():
                    wait_store(k - 2, b)

                _transpose_panel(pbuf, cbuf, b, PW, D, iot)
                store(k, b)

        wait_store(k_per_tile - 2, 0)
        wait_store(k_per_tile - 1, 1)

        @pl.when(wid < 5)
        def _():
            c0 = pl.multiple_of(tail0 + 128 * wid, 128)
            s0 = pl.multiple_of(tail0 // 2 + 64 * wid, 64)
            pltpu.async_copy(
                wT_hbm.at[:, pl.ds(c0, 128)], pbuf.at[0, :, pl.ds(0, 128)],
                psem.at[0],
            )
            pltpu.make_async_copy(
                wT_hbm.at[:, pl.ds(c0, 128)], pbuf.at[0, :, pl.ds(0, 128)],
                psem.at[0],
            ).wait()

            def tail_piece(width):
                _transpose_panel(pbuf, cbuf, 0, width, D, iot)

                pltpu.async_copy(
                    cbuf.at[0, pl.ds(0, width // 2)],
                    wc_hbm.at[pl.ds(s0, width // 2)],
                    wsem.at[0],
                )
                pltpu.make_async_copy(
                    cbuf.at[0, pl.ds(0, width // 2)],
                    wc_hbm.at[pl.ds(s0, width // 2)],
                    wsem.at[0],
                ).wait()

            @pl.when(wid < 4)
            def _():
                tail_piece(128)

            @pl.when(wid == 4)
            def _():
                tail_piece(64)

    @functools.partial(
        pl.kernel,
        out_type=jax.ShapeDtypeStruct(
            (NW * n_chunks, CHUNK // 2, 2 * D), jnp.float32
        ),
        mesh=mesh,
        scratch_types=[
            pltpu.VMEM((n_chunks, CHUNK), jnp.int32),
            pltpu.VMEM((NBUF, CHUNK), jnp.int32),
            pltpu.VMEM((NBUF, CHUNK, 2 * D), jnp.float32),
            pltpu.VMEM((NBUF, CHUNK // 2, 2 * D), jnp.float32),
            pltpu.SemaphoreType.DMA((NBUF,)),
            pltpu.SemaphoreType.DMA((NBUF,)),
        ],
        compiler_params=pltpu.CompilerParams(use_tc_tiling_on_sc=True),
    )
    def emb(idx_hbm, w_hbm, out_hbm, idx_v, sup_v, super_v, stage_v, gsem, wsem):
        wid = lax.axis_index("s") * NC + lax.axis_index("c")
        rbase = wid * n_chunks
        pltpu.sync_copy(idx_hbm.at[wid], idx_v)

        def compute_sup(c, b):
            for q in range(CHUNK // 16):
                sup_v[b, pl.ds(q * 16, 16)] = lax.shift_right_logical(
                    idx_v[c, pl.ds(q * 16, 16)], 1
                )

        for b in range(NBUF):
            compute_sup(b, b)
            pltpu.async_copy(w_hbm.at[sup_v.at[b]], super_v.at[b], gsem.at[b])

        @pl.loop(0, n_rounds)
        def body(g):
            for b in range(NBUF):
                cur = g * NBUF + b
                pltpu.make_async_copy(
                    w_hbm.at[sup_v.at[b]], super_v.at[b], gsem.at[b]
                ).wait()

                @pl.when(g > 0)
                def _():
                    pltpu.make_async_copy(
                        stage_v.at[b], out_hbm.at[rbase + cur - NBUF], wsem.at[b]
                    ).wait()

                @pl.loop(0, CHUNK // 16)
                def ext(k16):
                    hv = idx_v[cur, pl.ds(k16 * 16, 16)] & 1
                    for l in range(16):
                        hbase = hv[l] * D
                        k = k16 * 16 + l
                        k2, e = k16 * 8 + l // 2, l % 2
                        for p in range(D // 16):
                            stage_v[b, k2, pl.ds(e * D + p * 16, 16)] = (
                                super_v[b, k, pl.ds(hbase + p * 16, 16)]
                            )

                pltpu.async_copy(
                    stage_v.at[b], out_hbm.at[rbase + cur], wsem.at[b]
                )

                @pl.when(cur + NBUF < n_chunks)
                def _():
                    compute_sup(cur + NBUF, b)
                    pltpu.async_copy(
                        w_hbm.at[sup_v.at[b]], super_v.at[b], gsem.at[b]
                    )

        for b in range(NBUF):
            cur = n_chunks - NBUF + b
            pltpu.make_async_copy(
                stage_v.at[b], out_hbm.at[rbase + cur], wsem.at[b]
            ).wait()

    wc = compact(wT)
    out = emb(idx, wc)
    return out.reshape(B, S, D)


# speedup vs baseline: 1.3726x; 1.1528x over previous
import functools

import jax
import jax.numpy as jnp
from jax import lax
from jax.experimental import pallas as pl
from jax.experimental.pallas import tpu as pltpu
from jax.experimental.pallas import tpu_sc as plsc

NC = 2
NS = 16
NW = NC * NS
CHUNK = 128
NBUF = 2
PW = 256


def _transpose_panel(pbuf, cbuf, b, width, D, iot):
    @plsc.parallel_loop(0, width // 16, unroll=2)
    def grp(g):
        lo = g * 16
        colsv = lax.broadcast(lo, (16,)) + iot
        rows16 = lax.shift_right_logical(colsv, 1)
        colbase = (colsv & 1) * D
        for d in range(D):
            v = pbuf[b, d, pl.ds(lo, 16)]
            plsc.store_scatter(cbuf.at[b], [rows16, colbase + d], v)


def kernel(input_, weight):
    B, S = input_.shape
    V, D = weight.shape
    total = B * S
    assert total % (NW * CHUNK) == 0
    n_chunks = total // (NW * CHUNK)
    n_rounds = n_chunks // NBUF

    idx = input_.reshape(NW, n_chunks, CHUNK).astype(jnp.int32)
    wT = weight.T

    n_full = (V // PW) // NW * NW
    k_per_tile = n_full // NW
    tail0 = n_full * PW

    mesh = plsc.VectorSubcoreMesh(
        core_axis_name="c", subcore_axis_name="s", num_cores=NC, num_subcores=NS
    )

    @functools.partial(
        pl.kernel,
        out_type=jax.ShapeDtypeStruct((V // 2, 2 * D), jnp.float32),
        mesh=mesh,
        scratch_types=[
            pltpu.VMEM((2, D, PW), jnp.float32),
            pltpu.VMEM((2, PW // 2, 2 * D), jnp.float32),
            pltpu.SemaphoreType.DMA((2,)),
            pltpu.SemaphoreType.DMA((2,)),
        ],
        compiler_params=pltpu.CompilerParams(
            use_tc_tiling_on_sc=True, needs_layout_passes=False
        ),
    )
    def compact(wT_hbm, wc_hbm, pbuf, cbuf, psem, wsem):
        wid = lax.axis_index("s") * NC + lax.axis_index("c")
        iot = lax.iota(jnp.int32, 16)

        def col0(k):
            return pl.multiple_of((wid + 32 * k) * PW, PW)

        def sup0(k):
            return pl.multiple_of((wid + 32 * k) * (PW // 2), PW // 2)

        def load(k, b):
            pltpu.async_copy(
                wT_hbm.at[:, pl.ds(col0(k), PW)], pbuf.at[b], psem.at[b]
            )

        def wait_load(k, b):
            pltpu.make_async_copy(
                wT_hbm.at[:, pl.ds(col0(k), PW)], pbuf.at[b], psem.at[b]
            ).wait()

        def store(k, b):
            pltpu.async_copy(
                cbuf.at[b],
                wc_hbm.at[pl.ds(sup0(k), PW // 2)],
                wsem.at[b],
            )

        def wait_store(k, b):
            pltpu.make_async_copy(
                cbuf.at[b],
                wc_hbm.at[pl.ds(sup0(k), PW // 2)],
                wsem.at[b],
            ).wait()

        load(0, 0)

        @pl.loop(0, k_per_tile // 2)
        def body(jj):
            for koff in range(2):
                b = koff
                k = 2 * jj + koff
                wait_load(k, b)

                @pl.when(k < k_per_tile - 1)
                def _():
                    load(k + 1, 1 - b)

                @pl.when(jj > 0)
                def _():
                    wait_store(k - 2, b)

                _transpose_panel(pbuf, cbuf, b, PW, D, iot)
                store(k, b)

        wait_store(k_per_tile - 2, 0)
        wait_store(k_per_tile - 1, 1)

        @pl.when(wid < 5)
        def _():
            c0 = pl.multiple_of(tail0 + 128 * wid, 128)
            s0 = pl.multiple_of(tail0 // 2 + 64 * wid, 64)
            pltpu.async_copy(
                wT_hbm.at[:, pl.ds(c0, 128)], pbuf.at[0, :, pl.ds(0, 128)],
                psem.at[0],
            )
            pltpu.make_async_copy(
                wT_hbm.at[:, pl.ds(c0, 128)], pbuf.at[0, :, pl.ds(0, 128)],
                psem.at[0],
            ).wait()

            def tail_piece(width):
                _transpose_panel(pbuf, cbuf, 0, width, D, iot)

                pltpu.async_copy(
                    cbuf.at[0, pl.ds(0, width // 2)],
                    wc_hbm.at[pl.ds(s0, width // 2)],
                    wsem.at[0],
                )
                pltpu.make_async_copy(
                    cbuf.at[0, pl.ds(0, width // 2)],
                    wc_hbm.at[pl.ds(s0, width // 2)],
                    wsem.at[0],
                ).wait()

            @pl.when(wid < 4)
            def _():
                tail_piece(128)

            @pl.when(wid == 4)
            def _():
                tail_piece(64)

    @functools.partial(
        pl.kernel,
        out_type=jax.ShapeDtypeStruct(
            (NW * n_chunks, CHUNK // 2, 2 * D), jnp.float32
        ),
        mesh=mesh,
        scratch_types=[
            pltpu.VMEM((n_chunks, CHUNK), jnp.int32),
            pltpu.VMEM((NBUF, CHUNK), jnp.int32),
            pltpu.VMEM((NBUF, CHUNK, 2 * D), jnp.float32),
            pltpu.VMEM((NBUF, CHUNK // 2, 2 * D), jnp.float32),
            pltpu.SemaphoreType.DMA((NBUF,)),
            pltpu.SemaphoreType.DMA((NBUF,)),
        ],
        compiler_params=pltpu.CompilerParams(use_tc_tiling_on_sc=True),
    )
    def emb(idx_hbm, w_hbm, out_hbm, idx_v, sup_v, super_v, stage_v, gsem, wsem):
        wid = lax.axis_index("s") * NC + lax.axis_index("c")
        rbase = wid * n_chunks
        pltpu.sync_copy(idx_hbm.at[wid], idx_v)

        def compute_sup(c, b):
            for q in range(CHUNK // 16):
                sup_v[b, pl.ds(q * 16, 16)] = lax.shift_right_logical(
                    idx_v[c, pl.ds(q * 16, 16)], 1
                )

        for b in range(NBUF):
            compute_sup(b, b)
            pltpu.async_copy(w_hbm.at[sup_v.at[b]], super_v.at[b], gsem.at[b])

        @pl.loop(0, n_rounds)
        def body(g):
            for b in range(NBUF):
                cur = g * NBUF + b
                pltpu.make_async_copy(
                    w_hbm.at[sup_v.at[b]], super_v.at[b], gsem.at[b]
                ).wait()

                @pl.when(g > 0)
                def _():
                    pltpu.make_async_copy(
                        stage_v.at[b], out_hbm.at[rbase + cur - NBUF], wsem.at[b]
                    ).wait()

                @pl.loop(0, CHUNK // 16)
                def ext(k16):
                    hv = idx_v[cur, pl.ds(k16 * 16, 16)] & 1
                    for l in range(16):
                        hbase = hv[l] * D
                        k = k16 * 16 + l
                        k2, e = k16 * 8 + l // 2, l % 2
                        for p in range(D // 16):
                            stage_v[b, k2, pl.ds(e * D + p * 16, 16)] = (
                                super_v[b, k, pl.ds(hbase + p * 16, 16)]
                            )

                pltpu.async_copy(
                    stage_v.at[b], out_hbm.at[rbase + cur], wsem.at[b]
                )

                @pl.when(cur + NBUF < n_chunks)
                def _():
                    compute_sup(cur + NBUF, b)
                    pltpu.async_copy(
                        w_hbm.at[sup_v.at[b]], super_v.at[b], gsem.at[b]
                    )

        for b in range(NBUF):
            cur = n_chunks - NBUF + b
            pltpu.make_async_copy(
                stage_v.at[b], out_hbm.at[rbase + cur], wsem.at[b]
            ).wait()

    wc = compact(wT)
    out = emb(idx, wc)
    return out.reshape(B, S, D)


# speedup vs baseline: 2.3962x; 1.7457x over previous
import functools

import jax
import jax.numpy as jnp
from jax import lax
from jax.experimental import pallas as pl
from jax.experimental.pallas import tpu as pltpu
from jax.experimental.pallas import tpu_sc as plsc

NC = 2
NS = 16
NW = NC * NS
BK = 32


def kernel(input_, weight):
    B, S = input_.shape
    V, D = weight.shape
    total = B * S
    per_w = total // NW
    n_batches = per_w // BK
    n_rows = per_w // 128

    idx = input_.reshape(NW, n_rows, 128).astype(jnp.int32)
    w8 = weight.reshape(V // 8, 8, D)

    mesh = plsc.VectorSubcoreMesh(
        core_axis_name="c", subcore_axis_name="s", num_cores=NC, num_subcores=NS
    )

    @functools.partial(
        pl.kernel,
        out_type=jax.ShapeDtypeStruct(
            (NW * n_batches, BK // 2, 2 * D), jnp.float32
        ),
        mesh=mesh,
        scratch_types=[
            pltpu.VMEM((n_rows, 128), jnp.int32),
            pltpu.VMEM((2, BK, 8, D), jnp.float32),
            pltpu.VMEM((2, BK // 2, 2 * D), jnp.float32),
            pltpu.SemaphoreType.DMA((2,)),
            pltpu.SemaphoreType.DMA((2,)),
        ],
        compiler_params=pltpu.CompilerParams(
            use_tc_tiling_on_sc=True, needs_layout_passes=False
        ),
    )
    def emb(idx_hbm, w_hbm, out_hbm, idx_v, ring, stage_v, dsem, wsem):
        wid = lax.axis_index("s") * NC + lax.axis_index("c")
        pltpu.sync_copy(idx_hbm.at[wid], idx_v)

        def batch_vecs(b, g):
            row = lax.shift_right_logical(b, 2)
            col0 = (b & 3) * BK
            return idx_v[row, pl.ds(col0 + g * 16, 16)]

        def fire(b, h):
            for g in range(2):
                tv = lax.shift_right_logical(batch_vecs(b, g), 3)
                for l in range(16):
                    pltpu.async_copy(
                        w_hbm.at[tv[l]], ring.at[h, g * 16 + l], dsem.at[h]
                    )

        def drain(h):
            for j in range(BK):
                pltpu.make_async_copy(
                    w_hbm.at[0], ring.at[h, j], dsem.at[h]
                ).wait()

        def extract(b, h):
            for g in range(2):
                rv = batch_vecs(b, g) & 7
                for l in range(16):
                    r = rv[l]
                    j = g * 16 + l
                    j2, e = j // 2, j % 2
                    for p in range(D // 16):
                        stage_v[h, j2, pl.ds(e * D + p * 16, 16)] = (
                            ring[h, j, r, pl.ds(p * 16, 16)]
                        )

        def wait_write(b, h):
            pltpu.make_async_copy(
                stage_v.at[h], out_hbm.at[wid * n_batches + b], wsem.at[h]
            ).wait()

        fire(0, 0)

        @pl.loop(0, n_batches // 2)
        def body(jj):
            for bb in range(2):
                h = bb
                b = 2 * jj + bb

                @pl.when(b < n_batches - 1)
                def _():
                    fire(b + 1, 1 - h)

                drain(h)

                @pl.when(jj > 0)
                def _():
                    wait_write(b - 2, h)

                extract(b, h)
                pltpu.async_copy(
                    stage_v.at[h], out_hbm.at[wid * n_batches + b], wsem.at[h]
                )

        wait_write(n_batches - 2, 0)
        wait_write(n_batches - 1, 1)

    out = emb(idx, w8)
    return out.reshape(B, S, D)
